# baseline (speedup 1.0000x reference)
import functools

import jax
import jax.numpy as jnp
from jax import lax
from jax.experimental import pallas as pl
from jax.experimental.pallas import tpu as pltpu
from jax.experimental.pallas import tpu_sc as plsc

_B_ROWS = 1024
_B_COLS = 200
_B = _B_ROWS * _B_COLS
_D = 300
_NC = 2
_NS = 16
_NW = _NC * _NS
_CH = 128
_PER_W = _B // _NW
_NCH = _PER_W // _CH
_G = 16

_mesh = plsc.VectorSubcoreMesh(core_axis_name="c", subcore_axis_name="s")


@functools.partial(
    pl.kernel,
    mesh=_mesh,
    compiler_params=pltpu.CompilerParams(use_tc_tiling_on_sc=False),
    out_type=jax.ShapeDtypeStruct((_B, _D), jnp.float32),
    scratch_types=[
        pltpu.VMEM((_PER_W,), jnp.int32),
        pltpu.VMEM((_CH, _D), jnp.float32),
        pltpu.SemaphoreType.DMA,
    ],
)
def _emb_lookup(x_hbm, table_hbm, out_hbm, idx_v, rows_v, sem):
    wid = lax.axis_index("s") * _NC + lax.axis_index("c")
    base = wid * _PER_W
    pltpu.sync_copy(x_hbm.at[pl.ds(base, _PER_W)], idx_v)

    def chunk(c, carry):
        def grp(g, carry2):
            vec = idx_v[pl.ds(c * _CH + g * _G, _G)]
            for k in range(_G):
                v = vec[k]
                pltpu.async_copy(
                    table_hbm.at[pl.ds(v, 1), :],
                    rows_v.at[pl.ds(g * _G + k, 1), :],
                    sem,
                )
            return carry2

        lax.fori_loop(0, _CH // _G, grp, 0)
        def drain(_, carry2):
            pltpu.make_async_copy(
                table_hbm.at[pl.ds(0, 1), :], rows_v.at[pl.ds(0, 1), :], sem
            ).wait()
            return carry2

        lax.fori_loop(0, _CH, drain, 0)
        pltpu.sync_copy(rows_v, out_hbm.at[pl.ds(base + c * _CH, _CH)])
        return carry

    lax.fori_loop(0, _NCH, chunk, 0)


def kernel(x, table):
    idx = x.astype(jnp.int32).reshape(_B)
    out = _emb_lookup(idx, table)
    return out.reshape(_B_ROWS, _B_COLS, _D)


# speedup vs baseline: 1.0320x; 1.0320x over previous
import functools

import jax
import jax.numpy as jnp
from jax import lax
from jax.experimental import pallas as pl
from jax.experimental.pallas import tpu as pltpu
from jax.experimental.pallas import tpu_sc as plsc

_B_ROWS = 1024
_B_COLS = 200
_B = _B_ROWS * _B_COLS
_D = 300
_VP = 499997
_NC = 2
_NS = 16
_NW = _NC * _NS
_CH = 64
_PER_W = _B // _NW
_NCH = _PER_W // _CH
_G = 16

_mesh = plsc.VectorSubcoreMesh(core_axis_name="c", subcore_axis_name="s")


@functools.partial(
    pl.kernel,
    mesh=_mesh,
    compiler_params=pltpu.CompilerParams(use_tc_tiling_on_sc=False),
    out_type=jax.ShapeDtypeStruct((_B * _D,), jnp.float32),
    scratch_types=[
        pltpu.VMEM((_PER_W,), jnp.int32),
        pltpu.VMEM((_PER_W,), jnp.int32),
        pltpu.VMEM((_CH + 1, 2 * _D), jnp.float32),
        pltpu.VMEM((_CH * _D + _G,), jnp.float32),
        pltpu.SemaphoreType.DMA,
    ],
)
def _emb_lookup(x_hbm, table2_hbm, out_hbm, idx_v, pidx_v, pairs_v, packed_v, sem):
    wid = lax.axis_index("s") * _NC + lax.axis_index("c")
    base = wid * _PER_W
    pltpu.sync_copy(x_hbm.at[pl.ds(base, _PER_W)], idx_v)

    def mk_pidx(g, carry):
        vec = idx_v[pl.ds(g * _G, _G)]
        pidx_v[pl.ds(g * _G, _G)] = vec >> 1
        return carry

    lax.fori_loop(0, _PER_W // _G, mk_pidx, 0)

    def chunk(c, carry):
        pltpu.async_copy(
            table2_hbm.at[pidx_v.at[pl.ds(c * _CH, _CH)]],
            pairs_v.at[pl.ds(0, _CH)],
            sem,
        ).wait()

        def grp(g, carry2):
            vec = idx_v[pl.ds(c * _CH + g * _G, _G)]
            for k in range(_G):
                i = g * _G + k
                off = (vec[k] & 1) * _D
                for t in range(_D // _G + 1):
                    v16 = pairs_v[i, pl.ds(off + t * _G, _G)]
                    packed_v[pl.ds(i * _D + t * _G, _G)] = v16
            return carry2

        lax.fori_loop(0, _CH // _G, grp, 0)
        pltpu.sync_copy(
            packed_v.at[pl.ds(0, _CH * _D)],
            out_hbm.at[pl.ds((base + c * _CH) * _D, _CH * _D)],
        )
        return carry

    lax.fori_loop(0, _NCH, chunk, 0)


def kernel(x, table):
    idx = x.astype(jnp.int32).reshape(_B)
    table2 = table.reshape(_VP, 2 * _D)
    out = _emb_lookup(idx, table2)
    return out.reshape(_B_ROWS, _B_COLS, _D)


# speedup vs baseline: 1.0387x; 1.0066x over previous
import functools

import jax
import jax.numpy as jnp
from jax import lax
from jax.experimental import pallas as pl
from jax.experimental.pallas import tpu as pltpu
from jax.experimental.pallas import tpu_sc as plsc

_B_ROWS = 1024
_B_COLS = 200
_B = _B_ROWS * _B_COLS
_D = 300
_VP = 499997
_NC = 2
_NS = 16
_NW = _NC * _NS
_CH = 64
_PER_W = _B // _NW
_NCH = _PER_W // _CH
_G = 16

_mesh = plsc.VectorSubcoreMesh(core_axis_name="c", subcore_axis_name="s")


@functools.partial(
    pl.kernel,
    mesh=_mesh,
    compiler_params=pltpu.CompilerParams(use_tc_tiling_on_sc=False),
    out_type=jax.ShapeDtypeStruct((_B * _D,), jnp.float32),
    scratch_types=[
        pltpu.VMEM((_PER_W,), jnp.int32),
        pltpu.VMEM((_PER_W + _CH,), jnp.int32),
        pltpu.VMEM((_CH + 1, 2 * _D), jnp.float32),
        pltpu.VMEM((_CH + 1, 2 * _D), jnp.float32),
        pltpu.VMEM((_CH * _D + _G,), jnp.float32),
        pltpu.VMEM((_CH * _D + _G,), jnp.float32),
        pltpu.SemaphoreType.DMA,
        pltpu.SemaphoreType.DMA,
        pltpu.SemaphoreType.DMA,
        pltpu.SemaphoreType.DMA,
    ],
)
def _emb_lookup(x_hbm, table2_hbm, out_hbm, idx_v, pidx_v,
                pairs0, pairs1, packed0, packed1,
                gsem0, gsem1, wsem0, wsem1):
    wid = lax.axis_index("s") * _NC + lax.axis_index("c")
    base = wid * _PER_W
    pltpu.sync_copy(x_hbm.at[pl.ds(base, _PER_W)], idx_v)

    pairs = (pairs0, pairs1)
    packed = (packed0, packed1)
    gsem = (gsem0, gsem1)
    wsem = (wsem0, wsem1)

    def mk_pidx(g, carry):
        vec = idx_v[pl.ds(g * _G, _G)]
        pidx_v[pl.ds(g * _G, _G)] = vec >> 1
        return carry

    lax.fori_loop(0, _PER_W // _G, mk_pidx, 0)

    zeros = jnp.zeros((_G,), jnp.int32)
    for g in range(_CH // _G):
        pidx_v[pl.ds(_PER_W + g * _G, _G)] = zeros

    def issue_gather(c, b):
        pltpu.async_copy(
            table2_hbm.at[pidx_v.at[pl.ds(c * _CH, _CH)]],
            pairs[b].at[pl.ds(0, _CH)],
            gsem[b],
        )

    def wait_gather(b):
        pltpu.make_async_copy(
            table2_hbm.at[pl.ds(0, _CH)],
            pairs[b].at[pl.ds(0, _CH)],
            gsem[b],
        ).wait()

    def repack(c, b):
        def grp(g, carry2):
            vec = idx_v[pl.ds(c * _CH + g * _G, _G)]
            for k in range(_G):
                i = g * _G + k
                off = (vec[k] & 1) * _D
                for t in range(_D // _G + 1):
                    v16 = pairs[b][i, pl.ds(off + t * _G, _G)]
                    packed[b][pl.ds(i * _D + t * _G, _G)] = v16
            return carry2

        lax.fori_loop(0, _CH // _G, grp, 0)

    def issue_write(c, b):
        pltpu.async_copy(
            packed[b].at[pl.ds(0, _CH * _D)],
            out_hbm.at[pl.ds((base + c * _CH) * _D, _CH * _D)],
            wsem[b],
        )

    def wait_write(b):
        pltpu.make_async_copy(
            packed[b].at[pl.ds(0, _CH * _D)],
            out_hbm.at[pl.ds(base * _D, _CH * _D)],
            wsem[b],
        ).wait()

    issue_gather(0, 0)
    issue_gather(1, 1)
    wait_gather(0)
    repack(0, 0)
    issue_write(0, 0)
    wait_gather(1)
    issue_gather(2, 0)
    repack(1, 1)
    issue_write(1, 1)

    def steady(gidx, carry):
        for b in range(2):
            c = 2 * gidx + 2 + b
            wait_gather(b)
            issue_gather(c + 1, 1 - b)
            wait_write(b)
            repack(c, b)
            issue_write(c, b)
        return carry

    lax.fori_loop(0, (_NCH - 2) // 2, steady, 0)

    wait_write(0)
    wait_write(1)
    wait_gather(_NCH % 2)


def kernel(x, table):
    idx = x.astype(jnp.int32).reshape(_B)
    table2 = table.reshape(_VP, 2 * _D)
    out = _emb_lookup(idx, table2)
    return out.reshape(_B_ROWS, _B_COLS, _D)
